# baseline (speedup 1.0000x reference)
import functools

import jax
import jax.numpy as jnp
from jax import lax
from jax.experimental import pallas as pl
from jax.experimental.pallas import tpu as pltpu
from jax.experimental.pallas import tpu_sc as plsc

_B, _F = 4, 200000
_NC, _NS = 2, 16
_NW = _NC * _NS
_WPB = _NW // _B
_FPW = _F // _WPB
_NCH = 5
_CH = _FPW // _NCH
_CHP = ((_CH + 15) // 16) * 16
_GSTEPS = _CHP // 16


def _edges_body(face_hbm, out_hbm, fin, cols):
    wid = lax.axis_index("s") * _NC + lax.axis_index("c")
    b = wid // _WPB
    i0 = (wid % _WPB) * _FPW
    iota3 = lax.iota(jnp.int32, 16) * 3

    def chunk_body(ch, carry):
        base = i0 + ch * _CH
        pltpu.sync_copy(face_hbm.at[pl.ds(b * 3 * _F + 3 * base, 3 * _CH)],
                        fin.at[pl.ds(0, 3 * _CH)])

        def gather_body(j, c2):
            src = j * 48 + iota3
            dst = j * 16
            for c in range(3):
                vals = plsc.load_gather(fin, [src + c])
                cols[pl.ds(c * _CHP + dst, 16)] = vals
            return c2

        lax.fori_loop(0, _GSTEPS, gather_body, 0, unroll=2)

        obase = b * 6 * _F + base
        for c in range(3):
            col = cols.at[pl.ds(c * _CHP, _CH)]
            pltpu.sync_copy(col, out_hbm.at[pl.ds(obase + c * _F, _CH)])
            s1 = (c + 2) % 3
            pltpu.sync_copy(col, out_hbm.at[pl.ds(obase + 3 * _F + s1 * _F, _CH)])
        return carry

    lax.fori_loop(0, _NCH, chunk_body, 0)


_edges_call = functools.partial(
    pl.kernel,
    mesh=plsc.VectorSubcoreMesh(core_axis_name="c", subcore_axis_name="s"),
    out_type=jax.ShapeDtypeStruct((_B * 2 * 3 * _F,), jnp.int32),
    compiler_params=pltpu.CompilerParams(needs_layout_passes=False),
    scratch_types=[
        pltpu.VMEM((3 * _CH + 64,), jnp.int32),
        pltpu.VMEM((3 * _CHP,), jnp.int32),
    ],
)(_edges_body)


def kernel(x, mask, face):
    del mask
    face_flat = face.reshape(_B * 3 * _F)
    new_edges = _edges_call(face_flat).reshape(_B, 2, 3 * _F)
    return (x, face, new_edges)
